# Initial kernel scaffold; baseline (speedup 1.0000x reference)
import jax
import jax.numpy as jnp
from jax.experimental import pallas as pl


def kernel(x, edge_index, W1, W2):
    raise NotImplementedError("write your pallas kernel here")


# speedup vs baseline: 4.8290x; 4.8290x over previous
import functools

import jax
import jax.numpy as jnp
from jax import lax
from jax.experimental import pallas as pl
from jax.experimental.pallas import tpu as pltpu
from jax.experimental.pallas import tpu_sc as plsc

_N = 10000
_NP = 10240
_E = 160000
_D = 256
_H1 = 512
_OUT = 128

_NC = 2
_NS = 16
_CH = 128
_NCK = 80
_EPT = _NCK * _CH
_EPAD = _EPT * _NS
_RPT = _NP // _NS


def _make_agg(F: int):
    mesh = plsc.VectorSubcoreMesh(core_axis_name="c", subcore_axis_name="s")

    @functools.partial(
        pl.kernel,
        out_type=jax.ShapeDtypeStruct((_NC * _NP, F), jnp.float32),
        mesh=mesh,
        scratch_types=[
            pltpu.VMEM((2, 2, _CH), jnp.int32),
            pltpu.VMEM((2, _CH, F), jnp.float32),
            pltpu.VMEM_SHARED((_NP, F), jnp.float32),
            pltpu.SemaphoreType.DMA,
        ],
        compiler_params=pltpu.CompilerParams(use_tc_tiling_on_sc=False),
    )
    def agg(tab_hbm, idx_hbm, init_hbm, out_hbm,
            idx_v, rows_v, acc, gsem):
        cid = lax.axis_index("c")
        sid = lax.axis_index("s")
        row0 = sid * _RPT
        for c in range(0, _RPT, _CH):
            pltpu.sync_copy(init_hbm.at[pl.ds(cid * _NP + row0 + c, _CH)],
                            rows_v.at[0])
            pltpu.sync_copy(rows_v.at[0], acc.at[pl.ds(row0 + c, _CH)])
        plsc.subcore_barrier()
        ibase = (cid * _NS + sid) * _NCK

        def _fetch(jj, b):
            pltpu.sync_copy(idx_hbm.at[pl.ds(2 * (ibase + jj), 2)],
                            idx_v.at[b])
            pltpu.async_copy(tab_hbm.at[idx_v.at[b].at[0]],
                             rows_v.at[b], gsem)

        _fetch(0, 0)

        @pl.loop(0, _NCK, step=2)
        def _pair(j):
            for b in (0, 1):
                jj = j + b
                nxt = jj + 1

                @pl.when(nxt < _NCK)
                def _():
                    _fetch(nxt, 1 - b)

                pltpu.make_async_copy(tab_hbm.at[idx_v.at[b].at[0]],
                                      rows_v.at[b], gsem).wait()
                pltpu.sync_copy(rows_v.at[b], acc.at[idx_v.at[b].at[1]],
                                add=True)

        plsc.subcore_barrier()
        for c in range(0, _RPT, _CH):
            pltpu.sync_copy(acc.at[pl.ds(row0 + c, _CH)], rows_v.at[0])
            pltpu.sync_copy(rows_v.at[0],
                            out_hbm.at[pl.ds(cid * _NP + row0 + c, _CH)])

    return agg


_agg_d = _make_agg(_D // 2)
_agg_o = _make_agg(_OUT // 2)

_MB = 512


def _mm_body(x_ref, s_ref, w1_ref, w2_ref, p_ref, q_ref):
    xb = x_ref[...]
    hcat = jnp.concatenate([xb, s_ref[0], s_ref[1]], axis=1)
    h = jnp.maximum(
        jnp.dot(hcat, w1_ref[...], preferred_element_type=jnp.float32), 0.0)
    w2 = w2_ref[...]
    q = jnp.dot(h, w2[:_H1], preferred_element_type=jnp.float32)
    p = jnp.dot(h, w2[_H1:], preferred_element_type=jnp.float32)
    ho = _OUT // 2
    p_ref[0] = p[:, :ho]
    p_ref[1] = p[:, ho:]
    q_ref[0] = q[:, :ho]
    q_ref[1] = q[:, ho:]


_mm = pl.pallas_call(
    _mm_body,
    grid=(_NP // _MB,),
    in_specs=[
        pl.BlockSpec((_MB, _D), lambda i: (i, 0)),
        pl.BlockSpec((2, _MB, _D // 2), lambda i: (0, i, 0)),
        pl.BlockSpec((2 * _D, _H1), lambda i: (0, 0)),
        pl.BlockSpec((2 * _H1, _OUT), lambda i: (0, 0)),
    ],
    out_specs=[
        pl.BlockSpec((2, _MB, _OUT // 2), lambda i: (0, i, 0)),
        pl.BlockSpec((2, _MB, _OUT // 2), lambda i: (0, i, 0)),
    ],
    out_shape=[
        jax.ShapeDtypeStruct((2, _NP, _OUT // 2), jnp.float32),
        jax.ShapeDtypeStruct((2, _NP, _OUT // 2), jnp.float32),
    ],
)


def kernel(x, edge_index, W1, W2):
    i32 = jnp.int32
    src = edge_index[0].astype(i32)
    dst = edge_index[1].astype(i32)
    epad = _EPAD - _E
    srcp = jnp.concatenate([src, jnp.zeros((epad,), i32)])
    dstp = jnp.concatenate([dst, jnp.full((epad,), _N, i32)])
    dchunks = dstp.reshape(_NS * _NCK, _CH)
    idx_all = jnp.concatenate([
        jnp.stack([srcp.reshape(_NS * _NCK, _CH), dchunks], axis=1),
        jnp.stack([(srcp + _NP).reshape(_NS * _NCK, _CH), dchunks], axis=1),
    ], axis=0).reshape(2 * _NS * _NCK * 2, _CH)
    hd = _D // 2
    zrows = jnp.zeros((_NP - _N, hd), jnp.float32)
    tab1 = jnp.concatenate([x[:, :hd], zrows, x[:, hd:], zrows], axis=0)
    init1 = jnp.zeros((2 * _NP, hd), jnp.float32)
    sup = _agg_d(tab1, idx_all, init1).reshape(2, _NP, hd)
    xp = jnp.concatenate([x, jnp.zeros((_NP - _N, _D), jnp.float32)], axis=0)
    p_slab, q_slab = _mm(xp, sup, W1, W2)
    ho = _OUT // 2
    out2 = _agg_o(p_slab.reshape(2 * _NP, ho), idx_all,
                  q_slab.reshape(2 * _NP, ho))
    o = out2.reshape(2, _NP, ho)
    return jnp.concatenate([o[0, :_N], o[1, :_N]], axis=1)
